# baseline (speedup 1.0000x reference)
import functools

import jax
import jax.numpy as jnp
from jax import lax
from jax.experimental import pallas as pl
from jax.experimental.pallas import tpu as pltpu
from jax.experimental.pallas import tpu_sc as plsc

N_NODES = 10000
D = 128
E = 320000
NP = 10112
DUMMY = N_NODES
NC = 2
NS = 16
NW = NC * NS
K = 128
NCHUNK = 80
EPW = K * NCHUNK
ROWS_PER_TILE = NP // NS
DEGW = 128

_mesh = plsc.VectorSubcoreMesh(core_axis_name="c", subcore_axis_name="s")


@functools.partial(
    pl.kernel,
    mesh=_mesh,
    out_type=jax.ShapeDtypeStruct((NC, NP, DEGW), jnp.float32),
    scratch_types=[
        pltpu.VMEM((NCHUNK, K), jnp.int32),
        pltpu.VMEM((NCHUNK, K), jnp.int32),
        pltpu.VMEM((K, DEGW), jnp.float32),
        pltpu.VMEM_SHARED((NP, DEGW), jnp.float32),
        pltpu.SemaphoreType.DMA,
    ],
)
def _deg_kernel(row_hbm, col_hbm, ones_hbm, zeros_hbm, deg_hbm,
                midx_v, col_v, ones_v, acc_sh, sem):
    c = lax.axis_index("c")
    s = lax.axis_index("s")
    wid = s * NC + c
    pltpu.sync_copy(row_hbm.at[wid], midx_v)
    pltpu.sync_copy(col_hbm.at[wid], col_v)
    pltpu.sync_copy(ones_hbm, ones_v)
    pltpu.sync_copy(zeros_hbm, acc_sh.at[pl.ds(s * ROWS_PER_TILE, ROWS_PER_TILE)])

    def mloop(i, carry):
        for j in range(K // 16):
            sl = pl.ds(j * 16, 16)
            r = midx_v[i, sl]
            cc = col_v[i, sl]
            midx_v[i, sl] = jnp.where(r != cc, r, DUMMY)
        return carry

    lax.fori_loop(0, NCHUNK, mloop, 0)
    plsc.subcore_barrier()

    def sround(r, carry):
        i0 = r * 8
        for b in range(8):
            pltpu.async_copy(ones_v, acc_sh.at[midx_v.at[i0 + b]], sem, add=True)
        for b in range(8):
            pltpu.make_async_copy(ones_v, acc_sh.at[pl.ds(0, K)], sem).wait()
        return carry

    lax.fori_loop(0, NCHUNK // 8, sround, 0)
    plsc.subcore_barrier()
    sl = pl.ds(s * ROWS_PER_TILE, ROWS_PER_TILE)
    pltpu.sync_copy(acc_sh.at[sl], deg_hbm.at[c, sl])


C64 = 64
N64 = NCHUNK * 2
NHEX = N64 // 16


@functools.partial(
    pl.kernel,
    mesh=_mesh,
    out_type=jax.ShapeDtypeStruct((NC, NP, D), jnp.float32),
    scratch_types=[
        pltpu.VMEM((NCHUNK, K), jnp.int32),
        pltpu.VMEM((8, K), jnp.int32),
        pltpu.VMEM((8, K), jnp.int32),
        pltpu.VMEM((4, C64), jnp.int32),
        pltpu.VMEM((4, C64, D), jnp.float32),
        pltpu.VMEM_SHARED((NP, D), jnp.float32),
    ] + [pltpu.SemaphoreType.DMA] * 10,
)
def _prop_kernel(h_hbm, row_hbm, col_hbm, zeros_hbm, p_hbm,
                 col_v, rw0, rw1, ix_s, bufs, acc_sh,
                 g0, g1, g2, g3, s0, s1, s2, s3, r0, r1):
    c = lax.axis_index("c")
    s = lax.axis_index("s")
    wid = s * NC + c
    rw = (rw0, rw1)
    gs = (g0, g1, g2, g3)
    ss = (s0, s1, s2, s3)
    rs = (r0, r1)
    pltpu.sync_copy(zeros_hbm, acc_sh.at[pl.ds(s * ROWS_PER_TILE, ROWS_PER_TILE)])
    pltpu.sync_copy(col_hbm.at[wid], col_v)
    plsc.subcore_barrier()

    def col_ref(h, k):
        return col_v.at[8 * h + k // 2, pl.ds((k % 2) * C64, C64)]

    def gather(h, k, b):
        pltpu.async_copy(h_hbm.at[col_ref(h, k)], bufs.at[b], gs[b])

    def wait_gather(b):
        pltpu.make_async_copy(h_hbm.at[pl.ds(0, C64)], bufs.at[b], gs[b]).wait()

    def load_rows(h, p):
        pltpu.async_copy(row_hbm.at[wid, pl.ds(8 * h, 8)], rw[p], rs[p])

    def wait_rows(p):
        pltpu.make_async_copy(row_hbm.at[0, pl.ds(0, 8)], rw[p], rs[p]).wait()

    def scatter(b):
        pltpu.async_copy(bufs.at[b], acc_sh.at[ix_s.at[b]], ss[b], add=True)

    def wait_scatter(b):
        pltpu.make_async_copy(bufs.at[b], acc_sh.at[pl.ds(0, C64)], ss[b]).wait()

    load_rows(0, 0)
    gather(0, 0, 0)
    gather(0, 1, 1)

    def two_groups(t, carry):
        for p in range(2):
            h = 2 * t + p
            wait_rows(p)

            @pl.when(h + 1 < NHEX)
            def _():
                load_rows(h + 1, 1 - p)

            for k in range(16):
                b = k % 4
                i = 16 * h + k
                wait_gather(b)
                for j in range(C64 // 16):
                    sl = pl.ds((k % 2) * C64 + j * 16, 16)
                    rr = rw[p][k // 2, sl]
                    cc = col_v[8 * h + k // 2, sl]
                    ix_s[b, pl.ds(j * 16, 16)] = jnp.where(rr != cc, rr, DUMMY)
                scatter(b)

                b2 = (k + 2) % 4

                @pl.when(i >= 2)
                def _():
                    wait_scatter(b2)

                @pl.when(i + 2 < N64)
                def _():
                    gather(h + (k + 2) // 16, (k + 2) % 16, b2)

        return carry

    lax.fori_loop(0, NHEX // 2, two_groups, 0)
    wait_scatter((N64 - 2) % 4)
    wait_scatter((N64 - 1) % 4)
    plsc.subcore_barrier()
    sl = pl.ds(s * ROWS_PER_TILE, ROWS_PER_TILE)
    pltpu.sync_copy(acc_sh.at[sl], p_hbm.at[c, sl])


BLK = 1264


def _tc1_body(x_ref, w_ref, d0_ref, d1_ref, o_ref):
    deg = d0_ref[:, :1] + d1_ref[:, :1] + 1.0
    z = jnp.dot(x_ref[...], w_ref[...], preferred_element_type=jnp.float32)
    o_ref[...] = z * lax.rsqrt(deg)


def _make_combine(use_rsqrt):
    def body(p0_ref, p1_ref, c_ref, d0_ref, d1_ref, o_ref):
        deg = d0_ref[:, :1] + d1_ref[:, :1] + 1.0
        scale = lax.rsqrt(deg) if use_rsqrt else 1.0 / deg
        o_ref[...] = (p0_ref[...] + p1_ref[...] + c_ref[...]) * scale
    return body


_row_spec = pl.BlockSpec((BLK, D), lambda i: (i, 0))
_deg_spec = pl.BlockSpec((BLK, DEGW), lambda i: (i, 0))


def _tc1(xp, W, d0, d1):
    return pl.pallas_call(
        _tc1_body,
        grid=(NP // BLK,),
        in_specs=[_row_spec, pl.BlockSpec((D, D), lambda i: (0, 0)),
                  _deg_spec, _deg_spec],
        out_specs=_row_spec,
        out_shape=jax.ShapeDtypeStruct((NP, D), jnp.float32),
    )(xp, W, d0, d1)


def _combine(p0, p1, cc, d0, d1, use_rsqrt):
    return pl.pallas_call(
        _make_combine(use_rsqrt),
        grid=(NP // BLK,),
        in_specs=[_row_spec, _row_spec, _row_spec, _deg_spec, _deg_spec],
        out_specs=_row_spec,
        out_shape=jax.ShapeDtypeStruct((NP, D), jnp.float32),
    )(p0, p1, cc, d0, d1)


def kernel(edge_index, x, W):
    row = edge_index[0].astype(jnp.int32)
    col = edge_index[1].astype(jnp.int32)
    eprw = E // NW
    padw = EPW - eprw
    pad_rows = N_NODES + (jnp.arange(padw, dtype=jnp.int32) % (NP - N_NODES))
    rowp = jnp.concatenate(
        [row.reshape(NW, eprw), jnp.broadcast_to(pad_rows, (NW, padw))],
        axis=1).reshape(NW, NCHUNK, K)
    colp = jnp.concatenate(
        [col.reshape(NW, eprw), jnp.zeros((NW, padw), jnp.int32)],
        axis=1).reshape(NW, NCHUNK, K)
    xp = jnp.pad(x, ((0, NP - N_NODES), (0, 0)))
    ones_deg = jnp.ones((K, DEGW), jnp.float32)
    zeros_init = jnp.zeros((ROWS_PER_TILE, D), jnp.float32)

    degp = _deg_kernel(rowp, colp, ones_deg, zeros_init)
    d0 = degp[0]
    d1 = degp[1]
    t1 = _tc1(xp, W, d0, d1)
    p = _prop_kernel(t1, rowp, colp, zeros_init)
    t2 = _combine(p[0], p[1], t1, d0, d1, use_rsqrt=False)
    p2 = _prop_kernel(t2, rowp, colp, zeros_init)
    out = _combine(p2[0], p2[1], t2, d0, d1, use_rsqrt=True)
    return out[:N_NODES]


# speedup vs baseline: 1.0319x; 1.0319x over previous
import functools

import jax
import jax.numpy as jnp
from jax import lax
from jax.experimental import pallas as pl
from jax.experimental.pallas import tpu as pltpu
from jax.experimental.pallas import tpu_sc as plsc

N_NODES = 10000
D = 128
E = 320000
NP = 10112
DUMMY = N_NODES
NC = 2
NS = 16
NW = NC * NS
K = 128
NCHUNK = 80
EPW = K * NCHUNK
ROWS_PER_TILE = NP // NS
DEGW = 128

_mesh = plsc.VectorSubcoreMesh(core_axis_name="c", subcore_axis_name="s")


@functools.partial(
    pl.kernel,
    mesh=_mesh,
    out_type=jax.ShapeDtypeStruct((NC, NP, DEGW), jnp.float32),
    scratch_types=[
        pltpu.VMEM((NCHUNK, K), jnp.int32),
        pltpu.VMEM((NCHUNK, K), jnp.int32),
        pltpu.VMEM((K, DEGW), jnp.float32),
        pltpu.VMEM_SHARED((NP, DEGW), jnp.float32),
        pltpu.SemaphoreType.DMA,
    ],
)
def _deg_kernel(row_hbm, col_hbm, ones_hbm, zeros_hbm, deg_hbm,
                midx_v, col_v, ones_v, acc_sh, sem):
    c = lax.axis_index("c")
    s = lax.axis_index("s")
    wid = s * NC + c
    pltpu.sync_copy(row_hbm.at[wid], midx_v)
    pltpu.sync_copy(col_hbm.at[wid], col_v)
    pltpu.sync_copy(ones_hbm, ones_v)
    pltpu.sync_copy(zeros_hbm, acc_sh.at[pl.ds(s * ROWS_PER_TILE, ROWS_PER_TILE)])

    def mloop(i, carry):
        for j in range(K // 16):
            sl = pl.ds(j * 16, 16)
            r = midx_v[i, sl]
            cc = col_v[i, sl]
            midx_v[i, sl] = jnp.where(r != cc, r, DUMMY)
        return carry

    lax.fori_loop(0, NCHUNK, mloop, 0)
    plsc.subcore_barrier()

    def sround(r, carry):
        i0 = r * 8
        for b in range(8):
            pltpu.async_copy(ones_v, acc_sh.at[midx_v.at[i0 + b]], sem, add=True)
        for b in range(8):
            pltpu.make_async_copy(ones_v, acc_sh.at[pl.ds(0, K)], sem).wait()
        return carry

    lax.fori_loop(0, NCHUNK // 8, sround, 0)
    plsc.subcore_barrier()
    sl = pl.ds(s * ROWS_PER_TILE, ROWS_PER_TILE)
    pltpu.sync_copy(acc_sh.at[sl], deg_hbm.at[c, sl])


@functools.partial(
    pl.kernel,
    mesh=_mesh,
    out_type=jax.ShapeDtypeStruct((NC, NP, D), jnp.float32),
    scratch_types=[
        pltpu.VMEM((NCHUNK, K), jnp.int32),
        pltpu.VMEM((8, K), jnp.int32),
        pltpu.VMEM((8, K), jnp.int32),
        pltpu.VMEM((8, K), jnp.int32),
        pltpu.VMEM((8, K), jnp.int32),
        pltpu.VMEM((K, D), jnp.float32),
        pltpu.VMEM((K, D), jnp.float32),
        pltpu.VMEM_SHARED((NP, D), jnp.float32),
    ] + [pltpu.SemaphoreType.DMA] * 6,
)
def _prop_kernel(h_hbm, row_hbm, col_hbm, zeros_hbm, p_hbm,
                 col_v, rw0, rw1, ix0, ix1, buf0, buf1, acc_sh,
                 g0, g1, s0, s1, r0, r1):
    c = lax.axis_index("c")
    s = lax.axis_index("s")
    wid = s * NC + c
    rw = (rw0, rw1)
    ix = (ix0, ix1)
    buf = (buf0, buf1)
    gs = (g0, g1)
    ss = (s0, s1)
    rs = (r0, r1)
    pltpu.sync_copy(zeros_hbm, acc_sh.at[pl.ds(s * ROWS_PER_TILE, ROWS_PER_TILE)])
    pltpu.sync_copy(col_hbm.at[wid], col_v)
    plsc.subcore_barrier()

    def gather(i, b):
        pltpu.async_copy(h_hbm.at[col_v.at[i]], buf[b], gs[b])

    def wait_gather(b):
        pltpu.make_async_copy(h_hbm.at[pl.ds(0, K)], buf[b], gs[b]).wait()

    def load_rows(o, p):
        pltpu.async_copy(row_hbm.at[wid, pl.ds(8 * o, 8)], rw[p], rs[p])

    def wait_rows(p):
        pltpu.make_async_copy(row_hbm.at[0, pl.ds(0, 8)], rw[p], rs[p]).wait()

    def scatter(idx_ref, b):
        pltpu.async_copy(buf[b], acc_sh.at[idx_ref], ss[b], add=True)

    def wait_scatter(b):
        pltpu.make_async_copy(buf[b], acc_sh.at[pl.ds(0, K)], ss[b]).wait()

    NOCT = NCHUNK // 8
    load_rows(0, 0)
    gather(0, 0)
    gather(1, 1)

    def two_octets(t, carry):
        for p in range(2):
            o = 2 * t + p
            wait_rows(p)

            @pl.when(o + 1 < NOCT)
            def _():
                load_rows(o + 1, 1 - p)

            for k in range(8):
                for j in range(K // 16):
                    sl = pl.ds(j * 16, 16)
                    rr = rw[p][k, sl]
                    cc = col_v[8 * o + k, sl]
                    ix[p][k, sl] = jnp.where(rr != cc, rr, DUMMY)

            for k in range(8):
                b = k % 2
                i = 8 * o + k
                wait_gather(b)
                scatter(ix[p].at[k], b)
                wait_scatter(b)

                @pl.when(i + 2 < NCHUNK)
                def _():
                    gather(i + 2, b)

        return carry

    lax.fori_loop(0, NOCT // 2, two_octets, 0)
    plsc.subcore_barrier()
    sl = pl.ds(s * ROWS_PER_TILE, ROWS_PER_TILE)
    pltpu.sync_copy(acc_sh.at[sl], p_hbm.at[c, sl])


BLK = 1264


def _tc1_body(x_ref, w_ref, d0_ref, d1_ref, o_ref):
    deg = d0_ref[:, :1] + d1_ref[:, :1] + 1.0
    z = jnp.dot(x_ref[...], w_ref[...], preferred_element_type=jnp.float32)
    o_ref[...] = z * lax.rsqrt(deg)


def _make_combine(use_rsqrt):
    def body(p0_ref, p1_ref, c_ref, d0_ref, d1_ref, o_ref):
        deg = d0_ref[:, :1] + d1_ref[:, :1] + 1.0
        scale = lax.rsqrt(deg) if use_rsqrt else 1.0 / deg
        o_ref[...] = (p0_ref[...] + p1_ref[...] + c_ref[...]) * scale
    return body


_row_spec = pl.BlockSpec((BLK, D), lambda i: (i, 0))
_deg_spec = pl.BlockSpec((BLK, DEGW), lambda i: (i, 0))


def _tc1(xp, W, d0, d1):
    return pl.pallas_call(
        _tc1_body,
        grid=(NP // BLK,),
        in_specs=[_row_spec, pl.BlockSpec((D, D), lambda i: (0, 0)),
                  _deg_spec, _deg_spec],
        out_specs=_row_spec,
        out_shape=jax.ShapeDtypeStruct((NP, D), jnp.float32),
    )(xp, W, d0, d1)


def _combine(p0, p1, cc, d0, d1, use_rsqrt):
    return pl.pallas_call(
        _make_combine(use_rsqrt),
        grid=(NP // BLK,),
        in_specs=[_row_spec, _row_spec, _row_spec, _deg_spec, _deg_spec],
        out_specs=_row_spec,
        out_shape=jax.ShapeDtypeStruct((NP, D), jnp.float32),
    )(p0, p1, cc, d0, d1)


def kernel(edge_index, x, W):
    row = edge_index[0].astype(jnp.int32)
    col = edge_index[1].astype(jnp.int32)
    eprw = E // NW
    padw = EPW - eprw
    pad_rows = N_NODES + (jnp.arange(padw, dtype=jnp.int32) % (NP - N_NODES))
    rowp = jnp.concatenate(
        [row.reshape(NW, eprw), jnp.broadcast_to(pad_rows, (NW, padw))],
        axis=1).reshape(NW, NCHUNK, K)
    colp = jnp.concatenate(
        [col.reshape(NW, eprw), jnp.zeros((NW, padw), jnp.int32)],
        axis=1).reshape(NW, NCHUNK, K)
    xp = jnp.pad(x, ((0, NP - N_NODES), (0, 0)))
    ones_deg = jnp.ones((K, DEGW), jnp.float32)
    zeros_init = jnp.zeros((ROWS_PER_TILE, D), jnp.float32)

    degp = _deg_kernel(rowp, colp, ones_deg, zeros_init)
    d0 = degp[0]
    d1 = degp[1]
    t1 = _tc1(xp, W, d0, d1)
    p = _prop_kernel(t1, rowp, colp, zeros_init)
    t2 = _combine(p[0], p[1], t1, d0, d1, use_rsqrt=False)
    p2 = _prop_kernel(t2, rowp, colp, zeros_init)
    out = _combine(p2[0], p2[1], t2, d0, d1, use_rsqrt=True)
    return out[:N_NODES]


# speedup vs baseline: 1.0343x; 1.0024x over previous
import functools

import jax
import jax.numpy as jnp
from jax import lax
from jax.experimental import pallas as pl
from jax.experimental.pallas import tpu as pltpu
from jax.experimental.pallas import tpu_sc as plsc

N_NODES = 10000
D = 128
E = 320000
NP = 10112
DUMMY = N_NODES
NC = 2
NS = 16
NW = NC * NS
K = 128
NCHUNK = 80
EPW = K * NCHUNK
ROWS_PER_TILE = NP // NS
DEGW = 128

_mesh = plsc.VectorSubcoreMesh(core_axis_name="c", subcore_axis_name="s")


@functools.partial(
    pl.kernel,
    mesh=_mesh,
    out_type=jax.ShapeDtypeStruct((NC, NP, DEGW), jnp.float32),
    scratch_types=[
        pltpu.VMEM((NCHUNK, K), jnp.int32),
        pltpu.VMEM((NCHUNK, K), jnp.int32),
        pltpu.VMEM((K, DEGW), jnp.float32),
        pltpu.VMEM_SHARED((NP, DEGW), jnp.float32),
        pltpu.SemaphoreType.DMA,
        pltpu.SemaphoreType.DMA,
        pltpu.SemaphoreType.DMA,
        pltpu.SemaphoreType.DMA,
    ],
)
def _deg_kernel(row_hbm, col_hbm, ones_hbm, zeros_hbm, deg_hbm,
                midx_v, col_v, ones_v, acc_sh, sem, ra, rb, rc):
    c = lax.axis_index("c")
    s = lax.axis_index("s")
    wid = s * NC + c
    pltpu.async_copy(row_hbm.at[wid], midx_v, ra)
    pltpu.async_copy(col_hbm.at[wid], col_v, rb)
    pltpu.async_copy(ones_hbm, ones_v, rc)
    pltpu.sync_copy(zeros_hbm, acc_sh.at[pl.ds(s * ROWS_PER_TILE, ROWS_PER_TILE)])
    pltpu.make_async_copy(row_hbm.at[0], midx_v, ra).wait()
    pltpu.make_async_copy(col_hbm.at[0], col_v, rb).wait()
    pltpu.make_async_copy(ones_hbm, ones_v, rc).wait()

    def mloop(i, carry):
        for j in range(K // 16):
            sl = pl.ds(j * 16, 16)
            r = midx_v[i, sl]
            cc = col_v[i, sl]
            midx_v[i, sl] = jnp.where(r != cc, r, DUMMY)
        return carry

    lax.fori_loop(0, NCHUNK, mloop, 0)
    plsc.subcore_barrier()

    def sround(r, carry):
        i0 = r * 16
        for b in range(16):
            pltpu.async_copy(ones_v, acc_sh.at[midx_v.at[i0 + b]], sem, add=True)
        for b in range(16):
            pltpu.make_async_copy(ones_v, acc_sh.at[pl.ds(0, K)], sem).wait()
        return carry

    lax.fori_loop(0, NCHUNK // 16, sround, 0)
    plsc.subcore_barrier()
    sl = pl.ds(s * ROWS_PER_TILE, ROWS_PER_TILE)
    pltpu.sync_copy(acc_sh.at[sl], deg_hbm.at[c, sl])


@functools.partial(
    pl.kernel,
    mesh=_mesh,
    out_type=jax.ShapeDtypeStruct((NC, NP, D), jnp.float32),
    scratch_types=[
        pltpu.VMEM((NCHUNK, K), jnp.int32),
        pltpu.VMEM((8, K), jnp.int32),
        pltpu.VMEM((8, K), jnp.int32),
        pltpu.VMEM((8, K), jnp.int32),
        pltpu.VMEM((8, K), jnp.int32),
        pltpu.VMEM((K, D), jnp.float32),
        pltpu.VMEM((K, D), jnp.float32),
        pltpu.VMEM_SHARED((NP, D), jnp.float32),
    ] + [pltpu.SemaphoreType.DMA] * 7,
)
def _prop_kernel(h_hbm, row_hbm, col_hbm, zeros_hbm, p_hbm,
                 col_v, rw0, rw1, ix0, ix1, buf0, buf1, acc_sh,
                 g0, g1, s0, s1, r0, r1, rc):
    c = lax.axis_index("c")
    s = lax.axis_index("s")
    wid = s * NC + c
    rw = (rw0, rw1)
    ix = (ix0, ix1)
    buf = (buf0, buf1)
    gs = (g0, g1)
    ss = (s0, s1)
    rs = (r0, r1)
    pltpu.async_copy(col_hbm.at[wid], col_v, rc)
    pltpu.sync_copy(zeros_hbm, acc_sh.at[pl.ds(s * ROWS_PER_TILE, ROWS_PER_TILE)])
    pltpu.make_async_copy(col_hbm.at[0], col_v, rc).wait()
    plsc.subcore_barrier()

    def gather(i, b):
        pltpu.async_copy(h_hbm.at[col_v.at[i]], buf[b], gs[b])

    def wait_gather(b):
        pltpu.make_async_copy(h_hbm.at[pl.ds(0, K)], buf[b], gs[b]).wait()

    def load_rows(o, p):
        pltpu.async_copy(row_hbm.at[wid, pl.ds(8 * o, 8)], rw[p], rs[p])

    def wait_rows(p):
        pltpu.make_async_copy(row_hbm.at[0, pl.ds(0, 8)], rw[p], rs[p]).wait()

    def scatter(idx_ref, b):
        pltpu.async_copy(buf[b], acc_sh.at[idx_ref], ss[b], add=True)

    def wait_scatter(b):
        pltpu.make_async_copy(buf[b], acc_sh.at[pl.ds(0, K)], ss[b]).wait()

    NOCT = NCHUNK // 8
    load_rows(0, 0)
    gather(0, 0)
    gather(1, 1)

    def two_octets(t, carry):
        for p in range(2):
            o = 2 * t + p
            wait_rows(p)

            @pl.when(o + 1 < NOCT)
            def _():
                load_rows(o + 1, 1 - p)

            for k in range(8):
                for j in range(K // 16):
                    sl = pl.ds(j * 16, 16)
                    rr = rw[p][k, sl]
                    cc = col_v[8 * o + k, sl]
                    ix[p][k, sl] = jnp.where(rr != cc, rr, DUMMY)

            for k in range(8):
                b = k % 2
                i = 8 * o + k
                wait_gather(b)
                scatter(ix[p].at[k], b)
                wait_scatter(b)

                @pl.when(i + 2 < NCHUNK)
                def _():
                    gather(i + 2, b)

        return carry

    lax.fori_loop(0, NOCT // 2, two_octets, 0)
    plsc.subcore_barrier()
    sl = pl.ds(s * ROWS_PER_TILE, ROWS_PER_TILE)
    pltpu.sync_copy(acc_sh.at[sl], p_hbm.at[c, sl])


BLK = 1264


def _tc1_body(x_ref, w_ref, d0_ref, d1_ref, o_ref):
    deg = d0_ref[:, :1] + d1_ref[:, :1] + 1.0
    z = jnp.dot(x_ref[...], w_ref[...], preferred_element_type=jnp.float32)
    o_ref[...] = z * lax.rsqrt(deg)


def _make_combine(use_rsqrt):
    def body(p0_ref, p1_ref, c_ref, d0_ref, d1_ref, o_ref):
        deg = d0_ref[:, :1] + d1_ref[:, :1] + 1.0
        scale = lax.rsqrt(deg) if use_rsqrt else 1.0 / deg
        o_ref[...] = (p0_ref[...] + p1_ref[...] + c_ref[...]) * scale
    return body


_row_spec = pl.BlockSpec((BLK, D), lambda i: (i, 0))
_deg_spec = pl.BlockSpec((BLK, DEGW), lambda i: (i, 0))


def _tc1(xp, W, d0, d1):
    return pl.pallas_call(
        _tc1_body,
        grid=(NP // BLK,),
        in_specs=[_row_spec, pl.BlockSpec((D, D), lambda i: (0, 0)),
                  _deg_spec, _deg_spec],
        out_specs=_row_spec,
        out_shape=jax.ShapeDtypeStruct((NP, D), jnp.float32),
    )(xp, W, d0, d1)


def _combine(p0, p1, cc, d0, d1, use_rsqrt):
    return pl.pallas_call(
        _make_combine(use_rsqrt),
        grid=(NP // BLK,),
        in_specs=[_row_spec, _row_spec, _row_spec, _deg_spec, _deg_spec],
        out_specs=_row_spec,
        out_shape=jax.ShapeDtypeStruct((NP, D), jnp.float32),
    )(p0, p1, cc, d0, d1)


def kernel(edge_index, x, W):
    row = edge_index[0].astype(jnp.int32)
    col = edge_index[1].astype(jnp.int32)
    eprw = E // NW
    padw = EPW - eprw
    pad_rows = N_NODES + (jnp.arange(padw, dtype=jnp.int32) % (NP - N_NODES))
    rowp = jnp.concatenate(
        [row.reshape(NW, eprw), jnp.broadcast_to(pad_rows, (NW, padw))],
        axis=1).reshape(NW, NCHUNK, K)
    colp = jnp.concatenate(
        [col.reshape(NW, eprw), jnp.zeros((NW, padw), jnp.int32)],
        axis=1).reshape(NW, NCHUNK, K)
    xp = jnp.pad(x, ((0, NP - N_NODES), (0, 0)))
    ones_deg = jnp.ones((K, DEGW), jnp.float32)
    zeros_init = jnp.zeros((ROWS_PER_TILE, D), jnp.float32)

    degp = _deg_kernel(rowp, colp, ones_deg, zeros_init)
    d0 = degp[0]
    d1 = degp[1]
    t1 = _tc1(xp, W, d0, d1)
    p = _prop_kernel(t1, rowp, colp, zeros_init)
    t2 = _combine(p[0], p[1], t1, d0, d1, use_rsqrt=False)
    p2 = _prop_kernel(t2, rowp, colp, zeros_init)
    out = _combine(p2[0], p2[1], t2, d0, d1, use_rsqrt=True)
    return out[:N_NODES]
